# baseline (speedup 1.0000x reference)
import jax
import jax.numpy as jnp
from jax.experimental import pallas as pl
from jax.experimental.pallas import tpu as pltpu

_BLOCK_BL = 128


def _bcast_kernel(emb_ref, out_ref):
    out_ref[...] = jnp.broadcast_to(emb_ref[...][None], out_ref.shape)


def kernel(x, var_emb):
    B, L, D = x.shape
    d_model = var_emb.shape[1]
    BL = B * L
    emb = var_emb[:D]

    out = pl.pallas_call(
        _bcast_kernel,
        grid=(BL // _BLOCK_BL,),
        in_specs=[pl.BlockSpec((D, d_model), lambda i: (0, 0))],
        out_specs=pl.BlockSpec((_BLOCK_BL, D, d_model), lambda i: (i, 0, 0)),
        out_shape=jax.ShapeDtypeStruct((BL, D, d_model), var_emb.dtype),
        compiler_params=pltpu.CompilerParams(
            dimension_semantics=("arbitrary",),
            skip_device_barrier=True,
            disable_bounds_checks=True,
        ),
    )(emb)
    return out.reshape(B, L, D, d_model)


# speedup vs baseline: 1.0224x; 1.0224x over previous
import jax
import jax.numpy as jnp
from jax.experimental import pallas as pl
from jax.experimental.pallas import tpu as pltpu

_BLOCK_BL = 128


def _bcast_kernel(emb_ref, out_ref):
    out_ref[...] = jnp.broadcast_to(emb_ref[...][None], out_ref.shape)


def kernel(x, var_emb):
    B, L, D = x.shape
    d_model = var_emb.shape[1]
    BL = B * L
    emb = var_emb[:D]

    out = pl.pallas_call(
        _bcast_kernel,
        grid=(BL // _BLOCK_BL,),
        in_specs=[pl.BlockSpec(memory_space=pltpu.VMEM)],
        out_specs=pl.BlockSpec((_BLOCK_BL, D, d_model), lambda i: (i, 0, 0)),
        out_shape=jax.ShapeDtypeStruct((BL, D, d_model), var_emb.dtype),
        compiler_params=pltpu.CompilerParams(
            dimension_semantics=("arbitrary",),
            skip_device_barrier=True,
            disable_bounds_checks=True,
        ),
    )(emb)
    return out.reshape(B, L, D, d_model)
